# baseline (speedup 1.0000x reference)
import functools

import jax
import jax.numpy as jnp
from jax import lax
from jax.experimental import pallas as pl
from jax.experimental.pallas import tpu as pltpu
from jax.experimental.pallas import tpu_sc as plsc

N = 10000
F = 128
G = 16
PRE = 32
NCLS = 2

NC = 1
NS = 16
NW = NC * NS
RPW = 96
NSC = NW * RPW
NTC = N - NSC
NV = F // 16
SEGV = RPW // 16

BLK = NTC // 2
NBLK = NTC // BLK


def _sc_pool(x_hbm, seg_hbm, out_hbm, xbuf, segbuf, acc, sem):
    wid = lax.axis_index("s")
    start = NTC + wid * RPW
    xcopy = pltpu.async_copy(x_hbm.at[pl.ds(start, RPW)], xbuf, sem)
    pltpu.sync_copy(seg_hbm.at[pl.ds(start, RPW)], segbuf.at[pl.ds(0, RPW)])

    counts = [None] * (G + 1)
    counts[0] = 0
    counts[G] = RPW
    zero16 = jnp.zeros((16,), jnp.int32)
    one16 = jnp.ones((16,), jnp.int32)
    tv = [zero16] * (G - 1)
    for rg in range(SEGV):
        v = segbuf[pl.ds(rg * 16, 16)]
        for g in range(1, G):
            tv[g - 1] = tv[g - 1] + jnp.where(v < g, one16, zero16)
    for g in range(1, G):
        t = tv[g - 1]
        ssum = t[0]
        for i in range(1, 16):
            ssum = ssum + t[i]
        counts[g] = ssum

    xcopy.wait()

    zeros = jnp.zeros((16,), jnp.float32)
    for g in range(G):
        def inner(r, carry):
            return tuple(carry[j] + xbuf[r, pl.ds(j * 16, 16)]
                         for j in range(NV))
        res = lax.fori_loop(counts[g], counts[g + 1], inner, (zeros,) * NV)
        for j in range(NV):
            acc[g, pl.ds(j * 16, 16)] = res[j]

    pltpu.sync_copy(acc, out_hbm.at[wid])


_sc_pool_call = functools.partial(
    pl.kernel,
    out_type=jax.ShapeDtypeStruct((NW, G, F), jnp.float32),
    mesh=plsc.VectorSubcoreMesh(core_axis_name="c", subcore_axis_name="s", num_cores=1),
    scratch_types=[
        pltpu.VMEM((RPW, F), jnp.float32),
        pltpu.VMEM((RPW + 16,), jnp.int32),
        pltpu.VMEM((G, F), jnp.float32),
        pltpu.SemaphoreType.DMA,
    ],
)(_sc_pool)


def _tc_pool_kernel(x_ref, seg_ref, out_ref, acc_ref):
    i = pl.program_id(0)

    @pl.when(i == 0)
    def _init():
        acc_ref[...] = jnp.zeros_like(acc_ref)

    seg = seg_ref[0]
    gids = lax.broadcasted_iota(jnp.int32, (G, BLK), 0)
    onehot_t = (gids == seg).astype(jnp.float32)
    acc_ref[...] += lax.dot_general(
        onehot_t, x_ref[...],
        dimension_numbers=(((1,), (0,)), ((), ())),
        preferred_element_type=jnp.float32)

    @pl.when(i == NBLK - 1)
    def _finish():
        out_ref[...] = acc_ref[...]


def _tc_pool(x_tc, seg_tc):
    seg3 = seg_tc.reshape(NBLK, 1, BLK)
    return pl.pallas_call(
        _tc_pool_kernel,
        grid=(NBLK,),
        in_specs=[
            pl.BlockSpec((BLK, F), lambda i: (i, 0)),
            pl.BlockSpec((1, 1, BLK), lambda i: (i, 0, 0)),
        ],
        out_specs=pl.BlockSpec((G, F), lambda i: (0, 0)),
        out_shape=jax.ShapeDtypeStruct((G, F), jnp.float32),
        scratch_shapes=[pltpu.VMEM((G, F), jnp.float32)],
    )(x_tc, seg3)


def _combine_mlp_kernel(parts_ref, tcpool_ref, fc1w_ref, fc1b_ref, outw_ref,
                        outb_ref, out_ref):
    gid = lax.broadcasted_iota(jnp.int32, (G, NW * G), 0)
    cid = lax.broadcasted_iota(jnp.int32, (G, NW * G), 1)
    onehot_t = ((cid % G) == gid).astype(jnp.float32)
    pooled = lax.dot_general(
        onehot_t, parts_ref[...],
        dimension_numbers=(((1,), (0,)), ((), ())),
        preferred_element_type=jnp.float32) + tcpool_ref[...]
    h = lax.dot_general(
        pooled, fc1w_ref[...],
        dimension_numbers=(((1,), (0,)), ((), ())),
        preferred_element_type=jnp.float32) + fc1b_ref[...]
    logits = lax.dot_general(
        h, outw_ref[...],
        dimension_numbers=(((1,), (0,)), ((), ())),
        preferred_element_type=jnp.float32) + outb_ref[...]
    out_ref[...] = jax.nn.sigmoid(logits)


@jax.jit
def _run(x, seg, fc1_W, fc1_b, out_W, out_b):
    seg32 = seg.astype(jnp.int32)
    parts = _sc_pool_call(x, seg32)
    tc_pooled = _tc_pool(x[:NTC], seg32[:NTC])
    parts2 = parts.reshape(NW * G, F)
    return pl.pallas_call(
        _combine_mlp_kernel,
        in_specs=[
            pl.BlockSpec((NW * G, F), lambda: (0, 0)),
            pl.BlockSpec((G, F), lambda: (0, 0)),
            pl.BlockSpec((F, PRE), lambda: (0, 0)),
            pl.BlockSpec((1, PRE), lambda: (0, 0)),
            pl.BlockSpec((PRE, NCLS), lambda: (0, 0)),
            pl.BlockSpec((1, NCLS), lambda: (0, 0)),
        ],
        out_specs=pl.BlockSpec((G, NCLS), lambda: (0, 0)),
        out_shape=jax.ShapeDtypeStruct((G, NCLS), jnp.float32),
    )(parts2, tc_pooled, fc1_W, fc1_b.reshape(1, PRE), out_W,
      out_b.reshape(1, NCLS))


def kernel(x, edge_index, seg, kernel0, a_self0, a_neigh0, bias0,
           kernel1, a_self1, a_neigh1, bias1, fc1_W, fc1_b, out_W, out_b):
    return _run(x, seg, fc1_W, fc1_b, out_W, out_b)


# speedup vs baseline: 1.0136x; 1.0136x over previous
import functools

import jax
import jax.numpy as jnp
from jax import lax
from jax.experimental import pallas as pl
from jax.experimental.pallas import tpu as pltpu
from jax.experimental.pallas import tpu_sc as plsc

N = 10000
F = 128
G = 16
PRE = 32
NCLS = 2

NC = 1
NS = 16
NW = NC * NS
RPW = 192
NSC = NW * RPW
NTC = N - NSC
NV = F // 16
SEGV = RPW // 16

BLK = NTC // 2
NBLK = NTC // BLK


def _sc_pool(x_hbm, seg_hbm, out_hbm, xbuf, segbuf, acc, sem):
    wid = lax.axis_index("s")
    start = NTC + wid * RPW
    xcopy = pltpu.async_copy(x_hbm.at[pl.ds(start, RPW)], xbuf, sem)
    pltpu.sync_copy(seg_hbm.at[pl.ds(start, RPW)], segbuf.at[pl.ds(0, RPW)])

    counts = [None] * (G + 1)
    counts[0] = 0
    counts[G] = RPW
    zero16 = jnp.zeros((16,), jnp.int32)
    one16 = jnp.ones((16,), jnp.int32)
    tv = [zero16] * (G - 1)
    for rg in range(SEGV):
        v = segbuf[pl.ds(rg * 16, 16)]
        for g in range(1, G):
            tv[g - 1] = tv[g - 1] + jnp.where(v < g, one16, zero16)
    for g in range(1, G):
        t = tv[g - 1]
        ssum = t[0]
        for i in range(1, 16):
            ssum = ssum + t[i]
        counts[g] = ssum

    xcopy.wait()

    zeros = jnp.zeros((16,), jnp.float32)
    for g in range(G):
        def inner(r, carry):
            return tuple(carry[j] + xbuf[r, pl.ds(j * 16, 16)]
                         for j in range(NV))
        res = lax.fori_loop(counts[g], counts[g + 1], inner, (zeros,) * NV)
        for j in range(NV):
            acc[g, pl.ds(j * 16, 16)] = res[j]

    pltpu.sync_copy(acc, out_hbm.at[wid])


_sc_pool_call = functools.partial(
    pl.kernel,
    out_type=jax.ShapeDtypeStruct((NW, G, F), jnp.float32),
    mesh=plsc.VectorSubcoreMesh(core_axis_name="c", subcore_axis_name="s", num_cores=1),
    scratch_types=[
        pltpu.VMEM((RPW, F), jnp.float32),
        pltpu.VMEM((RPW + 16,), jnp.int32),
        pltpu.VMEM((G, F), jnp.float32),
        pltpu.SemaphoreType.DMA,
    ],
)(_sc_pool)


def _tc_pool_kernel(x_ref, seg_ref, out_ref, acc_ref):
    i = pl.program_id(0)

    @pl.when(i == 0)
    def _init():
        acc_ref[...] = jnp.zeros_like(acc_ref)

    seg = seg_ref[0]
    gids = lax.broadcasted_iota(jnp.int32, (G, BLK), 0)
    onehot_t = (gids == seg).astype(jnp.float32)
    acc_ref[...] += lax.dot_general(
        onehot_t, x_ref[...],
        dimension_numbers=(((1,), (0,)), ((), ())),
        preferred_element_type=jnp.float32)

    @pl.when(i == NBLK - 1)
    def _finish():
        out_ref[...] = acc_ref[...]


def _tc_pool(x_tc, seg_tc):
    seg3 = seg_tc.reshape(NBLK, 1, BLK)
    return pl.pallas_call(
        _tc_pool_kernel,
        grid=(NBLK,),
        in_specs=[
            pl.BlockSpec((BLK, F), lambda i: (i, 0)),
            pl.BlockSpec((1, 1, BLK), lambda i: (i, 0, 0)),
        ],
        out_specs=pl.BlockSpec((G, F), lambda i: (0, 0)),
        out_shape=jax.ShapeDtypeStruct((G, F), jnp.float32),
        scratch_shapes=[pltpu.VMEM((G, F), jnp.float32)],
    )(x_tc, seg3)


def _combine_mlp_kernel(parts_ref, tcpool_ref, fc1w_ref, fc1b_ref, outw_ref,
                        outb_ref, out_ref):
    gid = lax.broadcasted_iota(jnp.int32, (G, NW * G), 0)
    cid = lax.broadcasted_iota(jnp.int32, (G, NW * G), 1)
    onehot_t = ((cid % G) == gid).astype(jnp.float32)
    pooled = lax.dot_general(
        onehot_t, parts_ref[...],
        dimension_numbers=(((1,), (0,)), ((), ())),
        preferred_element_type=jnp.float32) + tcpool_ref[...]
    h = lax.dot_general(
        pooled, fc1w_ref[...],
        dimension_numbers=(((1,), (0,)), ((), ())),
        preferred_element_type=jnp.float32) + fc1b_ref[...]
    logits = lax.dot_general(
        h, outw_ref[...],
        dimension_numbers=(((1,), (0,)), ((), ())),
        preferred_element_type=jnp.float32) + outb_ref[...]
    out_ref[...] = jax.nn.sigmoid(logits)


@jax.jit
def _run(x, seg, fc1_W, fc1_b, out_W, out_b):
    seg32 = seg.astype(jnp.int32)
    parts = _sc_pool_call(x, seg32)
    tc_pooled = _tc_pool(x[:NTC], seg32[:NTC])
    parts2 = parts.reshape(NW * G, F)
    return pl.pallas_call(
        _combine_mlp_kernel,
        in_specs=[
            pl.BlockSpec((NW * G, F), lambda: (0, 0)),
            pl.BlockSpec((G, F), lambda: (0, 0)),
            pl.BlockSpec((F, PRE), lambda: (0, 0)),
            pl.BlockSpec((1, PRE), lambda: (0, 0)),
            pl.BlockSpec((PRE, NCLS), lambda: (0, 0)),
            pl.BlockSpec((1, NCLS), lambda: (0, 0)),
        ],
        out_specs=pl.BlockSpec((G, NCLS), lambda: (0, 0)),
        out_shape=jax.ShapeDtypeStruct((G, NCLS), jnp.float32),
    )(parts2, tc_pooled, fc1_W, fc1_b.reshape(1, PRE), out_W,
      out_b.reshape(1, NCLS))


def kernel(x, edge_index, seg, kernel0, a_self0, a_neigh0, bias0,
           kernel1, a_self1, a_neigh1, bias1, fc1_W, fc1_b, out_W, out_b):
    return _run(x, seg, fc1_W, fc1_b, out_W, out_b)
